# baseline (device time: 47505 ns/iter reference)
import jax
import jax.numpy as jnp
from jax import lax
from jax.experimental import pallas as pl
from jax.experimental.pallas import tpu as pltpu

M = 1024
HALF = M // 2


def kernel(dy, W):
    m, k = dy.shape

    def body(dy_ref, w_ref, out_ref,
             p_send, y_recv, r_buf, x_recv,
             send_sems, recv_sems):
        my_x = lax.axis_index("x")
        my_y = lax.axis_index("y")
        y_nbr = (my_x, 1 - my_y)
        x_nbr = (1 - my_x, my_y)

        barrier_sem = pltpu.get_barrier_semaphore()
        for nbr in (y_nbr, x_nbr):
            pl.semaphore_signal(
                barrier_sem, inc=1,
                device_id=nbr, device_id_type=pl.DeviceIdType.MESH,
            )
        pl.semaphore_wait(barrier_sem, 2)

        a = dy_ref[pl.ds(my_x * HALF, HALF), :].astype(jnp.bfloat16)
        w = w_ref[:, :].astype(jnp.bfloat16)
        p = lax.dot_general(
            a, w,
            dimension_numbers=(((1,), (1,)), ((), ())),
            preferred_element_type=jnp.float32,
        )
        p_send[:, :] = p.astype(jnp.bfloat16)

        e1 = pltpu.make_async_remote_copy(
            src_ref=p_send, dst_ref=y_recv,
            send_sem=send_sems.at[0], recv_sem=recv_sems.at[0],
            device_id=y_nbr, device_id_type=pl.DeviceIdType.MESH,
        )
        e1.start()
        e1.wait()

        r = p + y_recv[:, :].astype(jnp.float32)
        r_buf[:, :] = r.astype(jnp.bfloat16)
        out_ref[pl.ds(my_x * HALF, HALF), :] = r

        e2 = pltpu.make_async_remote_copy(
            src_ref=r_buf, dst_ref=x_recv,
            send_sem=send_sems.at[1], recv_sem=recv_sems.at[1],
            device_id=x_nbr, device_id_type=pl.DeviceIdType.MESH,
        )
        e2.start()
        e2.wait()

        out_ref[pl.ds((1 - my_x) * HALF, HALF), :] = (
            x_recv[:, :].astype(jnp.float32)
        )

    return pl.pallas_call(
        body,
        out_shape=jax.ShapeDtypeStruct((M, M), jnp.float32),
        in_specs=[
            pl.BlockSpec(memory_space=pltpu.VMEM),
            pl.BlockSpec(memory_space=pltpu.VMEM),
        ],
        out_specs=pl.BlockSpec(memory_space=pltpu.VMEM),
        scratch_shapes=[
            pltpu.VMEM((HALF, M), jnp.bfloat16),
            pltpu.VMEM((HALF, M), jnp.bfloat16),
            pltpu.VMEM((HALF, M), jnp.bfloat16),
            pltpu.VMEM((HALF, M), jnp.bfloat16),
            pltpu.SemaphoreType.DMA((2,)),
            pltpu.SemaphoreType.DMA((2,)),
        ],
        compiler_params=pltpu.CompilerParams(collective_id=0),
    )(dy, W)


# device time: 34221 ns/iter; 1.3882x vs baseline; 1.3882x over previous
import jax
import jax.numpy as jnp
from jax import lax
from jax.experimental import pallas as pl
from jax.experimental.pallas import tpu as pltpu

M = 1024
HALF = M // 2
C = 8
RC = HALF // C


def kernel(dy, W):
    def body(dy_ref, w_ref, out_ref,
             w_bf, p_send, y_recv, r_buf, x_recv,
             y_send_sems, y_recv_sems, x_send_sems, x_recv_sems):
        my_x = lax.axis_index("x")
        my_y = lax.axis_index("y")
        y_nbr = (my_x, 1 - my_y)
        x_nbr = (1 - my_x, my_y)

        barrier_sem = pltpu.get_barrier_semaphore()
        for nbr in (y_nbr, x_nbr):
            pl.semaphore_signal(
                barrier_sem, inc=1,
                device_id=nbr, device_id_type=pl.DeviceIdType.MESH,
            )
        pl.semaphore_wait(barrier_sem, 2)

        w_bf[:, :] = w_ref[:, :].astype(jnp.bfloat16)

        e1 = [None] * C
        e2 = [None] * C

        def rows(c):
            return pl.ds(c * RC, RC)

        def compute_and_send(c):
            a = dy_ref[pl.ds(my_x * HALF + c * RC, RC), :].astype(jnp.bfloat16)
            p = lax.dot_general(
                a, w_bf[:, :],
                dimension_numbers=(((1,), (1,)), ((), ())),
                preferred_element_type=jnp.float32,
            )
            p_send[rows(c), :] = p.astype(jnp.bfloat16)
            e1[c] = pltpu.make_async_remote_copy(
                src_ref=p_send.at[rows(c)], dst_ref=y_recv.at[rows(c)],
                send_sem=y_send_sems.at[c], recv_sem=y_recv_sems.at[c],
                device_id=y_nbr, device_id_type=pl.DeviceIdType.MESH,
            )
            e1[c].start()

        def reduce_and_send(c):
            e1[c].wait_recv()
            r = (p_send[rows(c), :].astype(jnp.float32)
                 + y_recv[rows(c), :].astype(jnp.float32))
            out_ref[pl.ds(my_x * HALF + c * RC, RC), :] = r
            r_buf[rows(c), :] = r.astype(jnp.bfloat16)
            e2[c] = pltpu.make_async_remote_copy(
                src_ref=r_buf.at[rows(c)], dst_ref=x_recv.at[rows(c)],
                send_sem=x_send_sems.at[c], recv_sem=x_recv_sems.at[c],
                device_id=x_nbr, device_id_type=pl.DeviceIdType.MESH,
            )
            e2[c].start()

        def store_remote(c):
            e2[c].wait_recv()
            out_ref[pl.ds((1 - my_x) * HALF + c * RC, RC), :] = (
                x_recv[rows(c), :].astype(jnp.float32)
            )

        for c in range(C):
            compute_and_send(c)
            if c >= 1:
                reduce_and_send(c - 1)
            if c >= 2:
                store_remote(c - 2)
        reduce_and_send(C - 1)
        store_remote(C - 2)
        store_remote(C - 1)

        for c in range(C):
            e1[c].wait_send()
            e2[c].wait_send()

    return pl.pallas_call(
        body,
        out_shape=jax.ShapeDtypeStruct((M, M), jnp.float32),
        in_specs=[
            pl.BlockSpec(memory_space=pltpu.VMEM),
            pl.BlockSpec(memory_space=pltpu.VMEM),
        ],
        out_specs=pl.BlockSpec(memory_space=pltpu.VMEM),
        scratch_shapes=[
            pltpu.VMEM(W.shape, jnp.bfloat16),
            pltpu.VMEM((HALF, M), jnp.bfloat16),
            pltpu.VMEM((HALF, M), jnp.bfloat16),
            pltpu.VMEM((HALF, M), jnp.bfloat16),
            pltpu.VMEM((HALF, M), jnp.bfloat16),
            pltpu.SemaphoreType.DMA((C,)),
            pltpu.SemaphoreType.DMA((C,)),
            pltpu.SemaphoreType.DMA((C,)),
            pltpu.SemaphoreType.DMA((C,)),
        ],
        compiler_params=pltpu.CompilerParams(collective_id=0),
    )(dy, W)


# device time: 33227 ns/iter; 1.4297x vs baseline; 1.0299x over previous
import jax
import jax.numpy as jnp
from jax import lax
from jax.experimental import pallas as pl
from jax.experimental.pallas import tpu as pltpu

M = 1024
HALF = M // 2
CD = 4
ND = M // CD
S = 2
CC = CD * S
NC = M // CC
K = 4096
RLAG = 2
SLAG = 3


def kernel(dy, W):
    def body(dy_hbm, w_hbm, out_hbm,
             a_f32, a_bf, w_stage, p_send, y_recv, r_buf, x_recv,
             o_mine, o_theirs,
             dy_sem, w_sems, o_sems,
             y_send_sems, y_recv_sems, x_send_sems, x_recv_sems):
        my_x = lax.axis_index("x")
        my_y = lax.axis_index("y")
        y_nbr = (my_x, 1 - my_y)
        x_nbr = (1 - my_x, my_y)

        dy_cp = pltpu.make_async_copy(
            dy_hbm.at[pl.ds(my_x * HALF, HALF), :], a_f32, dy_sem,
        )
        dy_cp.start()
        w_cps = [None] * CD
        w_cps[0] = pltpu.make_async_copy(
            w_hbm.at[pl.ds(0, ND), :], w_stage.at[0], w_sems.at[0],
        )
        w_cps[0].start()

        barrier_sem = pltpu.get_barrier_semaphore()
        for nbr in (y_nbr, x_nbr):
            pl.semaphore_signal(
                barrier_sem, inc=1,
                device_id=nbr, device_id_type=pl.DeviceIdType.MESH,
            )
        pl.semaphore_wait(barrier_sem, 2)

        dy_cp.wait()
        a_bf[:, :] = a_f32[:, :].astype(jnp.bfloat16)

        e1 = [None] * CC
        e2 = [None] * CC
        o_cps = [None] * (2 * CC)

        def dot_and_send(d):
            w_cps[d].wait()
            if d + 1 < CD:
                w_cps[d + 1] = pltpu.make_async_copy(
                    w_hbm.at[pl.ds((d + 1) * ND, ND), :],
                    w_stage.at[(d + 1) % 2],
                    w_sems.at[(d + 1) % 2],
                )
                w_cps[d + 1].start()
            w_bf = w_stage[d % 2, :, :].astype(jnp.bfloat16)
            p = lax.dot_general(
                a_bf[:, :], w_bf,
                dimension_numbers=(((1,), (1,)), ((), ())),
                preferred_element_type=jnp.float32,
            )
            for s in range(S):
                cc = d * S + s
                p_send[cc, :, :] = p[:, s * NC:(s + 1) * NC].astype(
                    jnp.bfloat16
                )
                e1[cc] = pltpu.make_async_remote_copy(
                    src_ref=p_send.at[cc], dst_ref=y_recv.at[cc],
                    send_sem=y_send_sems.at[cc], recv_sem=y_recv_sems.at[cc],
                    device_id=y_nbr, device_id_type=pl.DeviceIdType.MESH,
                )
                e1[cc].start()

        def reduce_and_send(cc):
            e1[cc].wait_recv()
            r = (p_send[cc, :, :].astype(jnp.float32)
                 + y_recv[cc, :, :].astype(jnp.float32))
            o_mine[cc, :, :] = r
            r_buf[cc, :, :] = r.astype(jnp.bfloat16)
            e2[cc] = pltpu.make_async_remote_copy(
                src_ref=r_buf.at[cc], dst_ref=x_recv.at[cc],
                send_sem=x_send_sems.at[cc], recv_sem=x_recv_sems.at[cc],
                device_id=x_nbr, device_id_type=pl.DeviceIdType.MESH,
            )
            e2[cc].start()
            o_cps[cc] = pltpu.make_async_copy(
                o_mine.at[cc],
                out_hbm.at[pl.ds(my_x * HALF, HALF), pl.ds(cc * NC, NC)],
                o_sems.at[cc],
            )
            o_cps[cc].start()

        def store_remote(cc):
            e2[cc].wait_recv()
            o_theirs[cc, :, :] = x_recv[cc, :, :].astype(jnp.float32)
            o_cps[CC + cc] = pltpu.make_async_copy(
                o_theirs.at[cc],
                out_hbm.at[pl.ds((1 - my_x) * HALF, HALF),
                           pl.ds(cc * NC, NC)],
                o_sems.at[CC + cc],
            )
            o_cps[CC + cc].start()

        for d in range(CD + SLAG + 1):
            if d < CD:
                dot_and_send(d)
            rd = d - RLAG
            if 0 <= rd < CD:
                for s in range(S):
                    reduce_and_send(rd * S + s)
            sd = d - SLAG
            if 0 <= sd < CD:
                for s in range(S):
                    store_remote(sd * S + s)

        for cc in range(CC):
            e1[cc].wait_send()
            e2[cc].wait_send()
        for i in range(2 * CC):
            o_cps[i].wait()

    return pl.pallas_call(
        body,
        out_shape=jax.ShapeDtypeStruct((M, M), jnp.float32),
        in_specs=[
            pl.BlockSpec(memory_space=pltpu.MemorySpace.HBM),
            pl.BlockSpec(memory_space=pltpu.MemorySpace.HBM),
        ],
        out_specs=pl.BlockSpec(memory_space=pltpu.MemorySpace.HBM),
        scratch_shapes=[
            pltpu.VMEM((HALF, K), jnp.float32),
            pltpu.VMEM((HALF, K), jnp.bfloat16),
            pltpu.VMEM((2, ND, K), jnp.float32),
            pltpu.VMEM((CC, HALF, NC), jnp.bfloat16),
            pltpu.VMEM((CC, HALF, NC), jnp.bfloat16),
            pltpu.VMEM((CC, HALF, NC), jnp.bfloat16),
            pltpu.VMEM((CC, HALF, NC), jnp.bfloat16),
            pltpu.VMEM((CC, HALF, NC), jnp.float32),
            pltpu.VMEM((CC, HALF, NC), jnp.float32),
            pltpu.SemaphoreType.DMA,
            pltpu.SemaphoreType.DMA((2,)),
            pltpu.SemaphoreType.DMA((2 * CC,)),
            pltpu.SemaphoreType.DMA((CC,)),
            pltpu.SemaphoreType.DMA((CC,)),
            pltpu.SemaphoreType.DMA((CC,)),
            pltpu.SemaphoreType.DMA((CC,)),
        ],
        compiler_params=pltpu.CompilerParams(collective_id=0),
    )(dy, W)
